# initial kernel scaffold (unmeasured)
import jax
import jax.numpy as jnp
from jax import lax
from jax.experimental import pallas as pl
from jax.experimental.pallas import tpu as pltpu

N_DEV = 4
TAPS = 4
HALO = TAPS - 1
CHUNK = 512


def kernel(x, k):
    B, S, C = x.shape
    assert S % CHUNK == 0
    n_chunks = S // CHUNK

    def body(x_ref, k_ref, out_ref, xe_ref, send_sem, recv_sem):
        my = lax.axis_index("i")
        left = lax.rem(my + N_DEV - 1, N_DEV)
        right = lax.rem(my + 1, N_DEV)

        barrier_sem = pltpu.get_barrier_semaphore()
        for nbr in (left, right):
            pl.semaphore_signal(
                barrier_sem,
                inc=1,
                device_id=(nbr,),
                device_id_type=pl.DeviceIdType.MESH,
            )
        pl.semaphore_wait(barrier_sem, 2)

        for c in range(n_chunks):
            o = c * CHUNK
            xe_ref[:, HALO + o : HALO + o + CHUNK, :] = x_ref[
                :, o : o + CHUNK, :
            ].astype(jnp.bfloat16)

        rdma = pltpu.make_async_remote_copy(
            src_ref=xe_ref.at[:, pl.ds(S, HALO), :],
            dst_ref=xe_ref.at[:, pl.ds(0, HALO), :],
            send_sem=send_sem,
            recv_sem=recv_sem,
            device_id=(right,),
            device_id_type=pl.DeviceIdType.MESH,
        )
        rdma.start()
        rdma.wait()

        @pl.when(my == 0)
        def _():
            xe_ref[:, 0:HALO, :] = jnp.zeros((B, HALO, C), jnp.bfloat16)

        for c in range(n_chunks):
            o = c * CHUNK
            acc = None
            for t in range(TAPS):
                xt = xe_ref[:, o + t : o + t + CHUNK, :].astype(jnp.float32)
                term = xt * k_ref[t, :][None, None, :]
                acc = term if acc is None else acc + term
            out_ref[:, o : o + CHUNK, :] = (
                acc * jax.nn.sigmoid(acc)
            ).astype(jnp.bfloat16)

    return pl.pallas_call(
        body,
        out_shape=jax.ShapeDtypeStruct((B, S, C), jnp.bfloat16),
        in_specs=[
            pl.BlockSpec(memory_space=pltpu.VMEM),
            pl.BlockSpec(memory_space=pltpu.VMEM),
        ],
        out_specs=pl.BlockSpec(memory_space=pltpu.VMEM),
        scratch_shapes=[
            pltpu.VMEM((B, S + HALO, C), jnp.bfloat16),
            pltpu.SemaphoreType.DMA,
            pltpu.SemaphoreType.DMA,
        ],
        compiler_params=pltpu.CompilerParams(collective_id=0),
    )(x, k)


# baseline (device time: 59437 ns/iter reference)
import jax
import jax.numpy as jnp
from jax import lax
from jax.experimental import pallas as pl
from jax.experimental.pallas import tpu as pltpu

N_DEV = 4
TAPS = 4
HALO = TAPS - 1
HALO_PAD = 8
SHIFT = HALO_PAD - HALO
CHUNK = 512


def kernel(x, k):
    B, S, C = x.shape
    assert S % CHUNK == 0
    n_chunks = S // CHUNK

    def body(x_ref, k_ref, out_ref, xe0_ref, send_sem, recv_sem):
        my = lax.axis_index("i")
        left = lax.rem(my + N_DEV - 1, N_DEV)
        right = lax.rem(my + 1, N_DEV)

        barrier_sem = pltpu.get_barrier_semaphore()
        for nbr in (left, right):
            pl.semaphore_signal(
                barrier_sem,
                inc=1,
                device_id=(nbr,),
                device_id_type=pl.DeviceIdType.MESH,
            )
        pl.semaphore_wait(barrier_sem, 2)

        rdma = pltpu.make_async_remote_copy(
            src_ref=x_ref.at[:, pl.ds(S - HALO_PAD, HALO_PAD), :],
            dst_ref=xe0_ref.at[:, pl.ds(0, HALO_PAD), :],
            send_sem=send_sem,
            recv_sem=recv_sem,
            device_id=(right,),
            device_id_type=pl.DeviceIdType.MESH,
        )
        rdma.start()

        xe0_ref[:, HALO_PAD : HALO_PAD + CHUNK, :] = x_ref[:, 0:CHUNK, :]

        def do_chunk(o, first):
            acc = None
            for t in range(TAPS):
                if first:
                    xt = xe0_ref[:, o + SHIFT + t : o + SHIFT + t + CHUNK, :]
                else:
                    xt = x_ref[:, o - HALO + t : o - HALO + t + CHUNK, :]
                term = xt * k_ref[t, :][None, None, :]
                acc = term if acc is None else acc + term
            out_ref[:, o : o + CHUNK, :] = (
                acc * jax.nn.sigmoid(acc)
            ).astype(jnp.bfloat16)

        for c in range(1, n_chunks):
            do_chunk(c * CHUNK, first=False)

        rdma.wait()

        @pl.when(my == 0)
        def _():
            xe0_ref[:, 0:HALO_PAD, :] = jnp.zeros((B, HALO_PAD, C), jnp.float32)

        do_chunk(0, first=True)

    return pl.pallas_call(
        body,
        out_shape=jax.ShapeDtypeStruct((B, S, C), jnp.bfloat16),
        in_specs=[
            pl.BlockSpec(memory_space=pltpu.VMEM),
            pl.BlockSpec(memory_space=pltpu.VMEM),
        ],
        out_specs=pl.BlockSpec(memory_space=pltpu.VMEM),
        scratch_shapes=[
            pltpu.VMEM((B, HALO_PAD + CHUNK, C), jnp.float32),
            pltpu.SemaphoreType.DMA,
            pltpu.SemaphoreType.DMA,
        ],
        compiler_params=pltpu.CompilerParams(
            collective_id=0,
            vmem_limit_bytes=100 * 1024 * 1024,
        ),
    )(x, k)


# device time: 43350 ns/iter; 1.3711x vs baseline; 1.3711x over previous
import jax
import jax.numpy as jnp
from jax import lax
from jax.experimental import pallas as pl
from jax.experimental.pallas import tpu as pltpu

N_DEV = 4
TAPS = 4
HALO = TAPS - 1
HALO_PAD = 8
SHIFT = HALO_PAD - HALO
CHUNK = 512
NSLOTS = 3


def kernel(x, k):
    B, S, C = x.shape
    assert S % CHUNK == 0
    n_chunks = S // CHUNK

    def body(x_hbm, k_ref, out_hbm, in_buf, out_buf, in_sems, out_sems,
             send_sem, recv_sem):
        my = lax.axis_index("i")
        left = lax.rem(my + N_DEV - 1, N_DEV)
        right = lax.rem(my + 1, N_DEV)

        barrier_sem = pltpu.get_barrier_semaphore()
        for nbr in (left, right):
            pl.semaphore_signal(
                barrier_sem,
                inc=1,
                device_id=(nbr,),
                device_id_type=pl.DeviceIdType.MESH,
            )
        pl.semaphore_wait(barrier_sem, 2)

        rdma = pltpu.make_async_remote_copy(
            src_ref=x_hbm.at[:, pl.ds(S - HALO_PAD, HALO_PAD), :],
            dst_ref=in_buf.at[0, :, pl.ds(0, HALO_PAD), :],
            send_sem=send_sem,
            recv_sem=recv_sem,
            device_id=(right,),
            device_id_type=pl.DeviceIdType.MESH,
        )
        rdma.start()

        def copy_in(c):
            o = c * CHUNK
            slot = c % NSLOTS
            cps = []
            for b in range(B):
                if c == 0:
                    cps.append(pltpu.make_async_copy(
                        x_hbm.at[b, pl.ds(0, CHUNK), :],
                        in_buf.at[slot, b, pl.ds(HALO_PAD, CHUNK), :],
                        in_sems.at[slot, b],
                    ))
                else:
                    cps.append(pltpu.make_async_copy(
                        x_hbm.at[b, pl.ds(o - HALO_PAD, CHUNK + HALO_PAD), :],
                        in_buf.at[slot, b, pl.ds(0, CHUNK + HALO_PAD), :],
                        in_sems.at[slot, b],
                    ))
            return cps

        def copy_out(c):
            o = c * CHUNK
            oslot = c % 2
            return [pltpu.make_async_copy(
                out_buf.at[oslot, b],
                out_hbm.at[b, pl.ds(o, CHUNK), :],
                out_sems.at[oslot, b],
            ) for b in range(B)]

        for cc_ in range(min(NSLOTS, n_chunks)):
            for cp in copy_in(cc_):
                cp.start()

        out_copies = {}
        for c in range(n_chunks):
            slot = c % NSLOTS
            for cp in copy_in(c):
                cp.wait()
            if c == 0:
                rdma.wait()

                @pl.when(my == 0)
                def _():
                    in_buf[0, :, 0:HALO_PAD, :] = jnp.zeros(
                        (B, HALO_PAD, C), jnp.float32
                    )

            if c >= 2:
                for cp in out_copies[c - 2]:
                    cp.wait()

            vb = in_buf[slot].astype(jnp.bfloat16)
            acc = None
            for t in range(TAPS):
                xt = vb[:, SHIFT + t : SHIFT + t + CHUNK, :]
                term = xt * k_ref[t, :][None, None, :].astype(jnp.bfloat16)
                acc = term if acc is None else acc + term
            out_buf[c % 2] = acc * jax.nn.sigmoid(acc)

            cps = copy_out(c)
            for cp in cps:
                cp.start()
            out_copies[c] = cps
            if c + NSLOTS < n_chunks:
                for cp in copy_in(c + NSLOTS):
                    cp.start()

        for c in range(n_chunks - 2, n_chunks):
            for cp in out_copies[c]:
                cp.wait()

    return pl.pallas_call(
        body,
        out_shape=jax.ShapeDtypeStruct((B, S, C), jnp.bfloat16),
        in_specs=[
            pl.BlockSpec(memory_space=pl.ANY),
            pl.BlockSpec(memory_space=pltpu.VMEM),
        ],
        out_specs=pl.BlockSpec(memory_space=pl.ANY),
        scratch_shapes=[
            pltpu.VMEM((NSLOTS, B, HALO_PAD + CHUNK, C), jnp.float32),
            pltpu.VMEM((2, B, CHUNK, C), jnp.bfloat16),
            pltpu.SemaphoreType.DMA((NSLOTS, 2)),
            pltpu.SemaphoreType.DMA((2, 2)),
            pltpu.SemaphoreType.DMA,
            pltpu.SemaphoreType.DMA,
        ],
        compiler_params=pltpu.CompilerParams(
            collective_id=0,
            vmem_limit_bytes=100 * 1024 * 1024,
        ),
    )(x, k)


# device time: 42881 ns/iter; 1.3861x vs baseline; 1.0109x over previous
import jax
import jax.numpy as jnp
from jax import lax
from jax.experimental import pallas as pl
from jax.experimental.pallas import tpu as pltpu

N_DEV = 4
TAPS = 4
HALO = TAPS - 1
HALO_PAD = 8
SHIFT = HALO_PAD - HALO
CHUNK = 512
NSLOTS = 3


def kernel(x, k):
    B, S, C = x.shape
    assert S % CHUNK == 0
    n_chunks = S // CHUNK

    def body(x_hbm, k_ref, out_hbm, in_buf, out_buf, in_sems, out_sems,
             send_sem, recv_sem):
        my = lax.axis_index("i")
        left = lax.rem(my + N_DEV - 1, N_DEV)
        right = lax.rem(my + 1, N_DEV)

        barrier_sem = pltpu.get_barrier_semaphore()
        for nbr in (left, right):
            pl.semaphore_signal(
                barrier_sem,
                inc=1,
                device_id=(nbr,),
                device_id_type=pl.DeviceIdType.MESH,
            )
        pl.semaphore_wait(barrier_sem, 2)

        rdma = pltpu.make_async_remote_copy(
            src_ref=x_hbm.at[:, pl.ds(S - HALO_PAD, HALO_PAD), :],
            dst_ref=in_buf.at[0, :, pl.ds(0, HALO_PAD), :],
            send_sem=send_sem,
            recv_sem=recv_sem,
            device_id=(right,),
            device_id_type=pl.DeviceIdType.MESH,
        )
        rdma.start()

        def copy_in(c):
            o = c * CHUNK
            slot = c % NSLOTS
            cps = []
            for b in range(B):
                if c == 0:
                    cps.append(pltpu.make_async_copy(
                        x_hbm.at[b, pl.ds(0, CHUNK), :],
                        in_buf.at[slot, b, pl.ds(HALO_PAD, CHUNK), :],
                        in_sems.at[slot, b],
                    ))
                else:
                    cps.append(pltpu.make_async_copy(
                        x_hbm.at[b, pl.ds(o - HALO_PAD, CHUNK + HALO_PAD), :],
                        in_buf.at[slot, b, pl.ds(0, CHUNK + HALO_PAD), :],
                        in_sems.at[slot, b],
                    ))
            return cps

        def copy_out(c):
            o = c * CHUNK
            oslot = c % 2
            return [pltpu.make_async_copy(
                out_buf.at[oslot, b],
                out_hbm.at[b, pl.ds(o, CHUNK), :],
                out_sems.at[oslot, b],
            ) for b in range(B)]

        for cc_ in range(min(NSLOTS, n_chunks)):
            for cp in copy_in(cc_):
                cp.start()

        out_copies = {}
        for c in range(n_chunks):
            slot = c % NSLOTS
            for cp in copy_in(c):
                cp.wait()
            if c == 0:
                rdma.wait()

                @pl.when(my == 0)
                def _():
                    in_buf[0, :, 0:HALO_PAD, :] = jnp.zeros(
                        (B, HALO_PAD, C), jnp.float32
                    )

            if c >= 2:
                for cp in out_copies[c - 2]:
                    cp.wait()

            kb = [k_ref[t, :][None, None, :].astype(jnp.bfloat16)
                  for t in range(TAPS)]
            va = in_buf[slot].astype(jnp.bfloat16)
            v1 = pltpu.roll(va, 1, 1)
            p = va * kb[3] + v1 * kb[2]
            q = va * kb[1] + v1 * kb[0]
            acc = (p + pltpu.roll(q, 2, 1))[
                :, HALO_PAD : HALO_PAD + CHUNK, :
            ]
            out_buf[c % 2] = acc * jax.nn.sigmoid(acc)

            cps = copy_out(c)
            for cp in cps:
                cp.start()
            out_copies[c] = cps
            if c + NSLOTS < n_chunks:
                for cp in copy_in(c + NSLOTS):
                    cp.start()

        for c in range(n_chunks - 2, n_chunks):
            for cp in out_copies[c]:
                cp.wait()

    return pl.pallas_call(
        body,
        out_shape=jax.ShapeDtypeStruct((B, S, C), jnp.bfloat16),
        in_specs=[
            pl.BlockSpec(memory_space=pl.ANY),
            pl.BlockSpec(memory_space=pltpu.VMEM),
        ],
        out_specs=pl.BlockSpec(memory_space=pl.ANY),
        scratch_shapes=[
            pltpu.VMEM((NSLOTS, B, HALO_PAD + CHUNK, C), jnp.float32),
            pltpu.VMEM((2, B, CHUNK, C), jnp.bfloat16),
            pltpu.SemaphoreType.DMA((NSLOTS, 2)),
            pltpu.SemaphoreType.DMA((2, 2)),
            pltpu.SemaphoreType.DMA,
            pltpu.SemaphoreType.DMA,
        ],
        compiler_params=pltpu.CompilerParams(
            collective_id=0,
            vmem_limit_bytes=100 * 1024 * 1024,
        ),
    )(x, k)
